# baseline (speedup 1.0000x reference)
import functools

import jax
import jax.numpy as jnp
from jax import lax
from jax.experimental import pallas as pl
from jax.experimental.pallas import tpu as pltpu
from jax.experimental.pallas import tpu_sc as plsc

CH = 512
PAD = 8
CHP = CH + PAD


def _sc_partial_sums(lengths, indices, table):
    B = lengths.shape[0]
    TOT = indices.shape[0]
    NHID = table.shape[1]
    MAXLEN = TOT // B
    max_half = MAXLEN
    n_chunks = (max_half + CH - 1) // CH

    mesh = plsc.VectorSubcoreMesh(core_axis_name="c", subcore_axis_name="s", num_cores=1)

    @functools.partial(
        pl.kernel,
        mesh=mesh,
        compiler_params=pltpu.CompilerParams(use_tc_tiling_on_sc=False),
        out_type=jax.ShapeDtypeStruct((1, B, NHID), jnp.float32),
        scratch_types=[
            pltpu.VMEM((B,), jnp.int32),
            pltpu.VMEM((CHP,), jnp.int32),
            pltpu.VMEM((CHP, NHID), jnp.float32),
            pltpu.VMEM((NHID,), jnp.float32),
            pltpu.SemaphoreType.DMA,
        ],
    )
    def k(len_hbm, idx_hbm, tab_hbm, out_hbm, len_v, idx_v, rows_v, acc_v, sem):
        cid = lax.axis_index("c")
        sid = lax.axis_index("s")

        pltpu.sync_copy(len_hbm, len_v)
        lv = len_v[...]
        seg_start = jnp.int32(0)
        seg_len = jnp.int32(0)
        run = jnp.int32(0)
        for j in range(B):
            lj = lv[j]
            seg_start = jnp.where(sid == j, run, seg_start)
            seg_len = jnp.where(sid == j, lj, seg_len)
            run = run + lj
        my_start = seg_start
        my_count = seg_len

        zero16 = jnp.zeros((16,), jnp.float32)
        for j in range(NHID // 16):
            acc_v[pl.ds(16 * j, 16)] = zero16

        for k_idx in range(n_chunks):
            cnt = jnp.clip(my_count - k_idx * CH, 0, CH)

            @pl.when(cnt > 0)
            def _():
                g0 = my_start + k_idx * CH
                a0 = jnp.minimum(g0, TOT - CHP)
                a0 = pl.multiple_of((a0 // 8) * 8, 8)
                pad = g0 - a0
                pltpu.sync_copy(idx_hbm.at[pl.ds(a0, CHP)], idx_v)
                cps = []
                for i in range(CH // 128):
                    cps.append(pltpu.async_copy(
                        tab_hbm.at[idx_v.at[pl.ds(128 * i, 128)]],
                        rows_v.at[pl.ds(128 * i, 128)], sem))
                cps.append(pltpu.async_copy(
                    tab_hbm.at[idx_v.at[pl.ds(CH, PAD)]],
                    rows_v.at[pl.ds(CH, PAD)], sem))
                for cp in cps:
                    cp.wait()

                b_lo = (pad // 8) * 8
                b_hi = ((pad + cnt + 7) // 8) * 8

                def zero_row(t, carry):
                    for j in range(NHID // 16):
                        rows_v[t, pl.ds(16 * j, 16)] = zero16
                    return carry

                lax.fori_loop(b_lo, pad, zero_row, 0)
                lax.fori_loop(pad + cnt, b_hi, zero_row, 0)

                def blk(i, carry):
                    a0c, a1c, a2c, a3c = carry
                    base = b_lo + i * 8
                    for r in range(8):
                        row = base + r
                        a0c = a0c + rows_v[row, pl.ds(0, 16)]
                        a1c = a1c + rows_v[row, pl.ds(16, 16)]
                        a2c = a2c + rows_v[row, pl.ds(32, 16)]
                        a3c = a3c + rows_v[row, pl.ds(48, 16)]
                    return a0c, a1c, a2c, a3c

                accs = lax.fori_loop(0, (b_hi - b_lo) // 8, blk,
                                     (zero16, zero16, zero16, zero16))
                for j in range(NHID // 16):
                    sl = pl.ds(16 * j, 16)
                    acc_v[sl] = acc_v[sl] + accs[j]

        pltpu.sync_copy(acc_v, out_hbm.at[cid, sid])

    return k(lengths, indices, table)


def _tc_finalize(partials, lengths, W, b):
    B = lengths.shape[0]
    NC = W.shape[0]

    def body(p_ref, l_ref, w_ref, b_ref, o_ref):
        sums = p_ref[0]
        lv = l_ref[0].astype(jnp.float32)
        inv = jnp.where(lv > 0, 1.0 / jnp.maximum(lv, 1.0), 0.0)
        means = sums * inv[:, None]
        t = jnp.tanh(means)
        o_ref[...] = lax.dot_general(
            t, w_ref[...], (((1,), (1,)), ((), ())),
            preferred_element_type=jnp.float32) + b_ref[...]

    return pl.pallas_call(
        body,
        out_shape=jax.ShapeDtypeStruct((B, NC), jnp.float32),
    )(partials, lengths.reshape(1, B), W, b.reshape(1, NC))


def kernel(lengths, indices, table, W, b):
    partials = _sc_partial_sums(lengths, indices, table)
    return _tc_finalize(partials, lengths, W, b)


# speedup vs baseline: 1.1258x; 1.1258x over previous
import functools

import jax
import jax.numpy as jnp
from jax import lax
from jax.experimental import pallas as pl
from jax.experimental.pallas import tpu as pltpu
from jax.experimental.pallas import tpu_sc as plsc

CH = 512
PAD = 8
CHP = CH + PAD


def _sc_partial_sums(lengths, indices, table_p, nhid):
    B = lengths.shape[0]
    TOT = indices.shape[0]
    NROW = table_p.shape[1]
    MAXLEN = TOT // B
    max_half = (MAXLEN + 1) // 2
    n_chunks = (max_half + CH - 1) // CH

    mesh = plsc.VectorSubcoreMesh(core_axis_name="c", subcore_axis_name="s")

    @functools.partial(
        pl.kernel,
        mesh=mesh,
        out_type=jax.ShapeDtypeStruct((2, B, nhid), jnp.float32),
        scratch_types=[
            pltpu.VMEM((B,), jnp.int32),
            pltpu.VMEM((CHP,), jnp.int32),
            pltpu.VMEM((CHP, NROW), jnp.float32),
            pltpu.VMEM((nhid,), jnp.float32),
            pltpu.SemaphoreType.DMA,
        ],
    )
    def k(len_hbm, idx_hbm, tab_hbm, out_hbm, len_v, idx_v, rows_v, acc_v, sem):
        cid = lax.axis_index("c")
        sid = lax.axis_index("s")

        pltpu.sync_copy(len_hbm, len_v)
        lv = len_v[...]
        seg_start = jnp.int32(0)
        seg_len = jnp.int32(0)
        run = jnp.int32(0)
        for j in range(B):
            lj = lv[j]
            seg_start = jnp.where(sid == j, run, seg_start)
            seg_len = jnp.where(sid == j, lj, seg_len)
            run = run + lj
        half0 = seg_len // 2
        my_start = seg_start + jnp.where(cid == 0, 0, half0)
        my_count = jnp.where(cid == 0, half0, seg_len - half0)

        zero16 = jnp.zeros((16,), jnp.float32)
        for j in range(nhid // 16):
            acc_v[pl.ds(16 * j, 16)] = zero16

        for k_idx in range(n_chunks):
            cnt = jnp.clip(my_count - k_idx * CH, 0, CH)

            @pl.when(cnt > 0)
            def _():
                g0 = my_start + k_idx * CH
                a0 = jnp.minimum(g0, TOT - CHP)
                a0 = pl.multiple_of((a0 // 8) * 8, 8)
                pad = g0 - a0
                pltpu.sync_copy(idx_hbm.at[pl.ds(a0, CHP)], idx_v)
                cps = []
                for i in range(CH // 128):
                    cps.append(pltpu.async_copy(
                        tab_hbm.at[idx_v.at[pl.ds(128 * i, 128)]],
                        rows_v.at[pl.ds(128 * i, 128)], sem))
                cps.append(pltpu.async_copy(
                    tab_hbm.at[idx_v.at[pl.ds(CH, PAD)]],
                    rows_v.at[pl.ds(CH, PAD)], sem))
                for cp in cps:
                    cp.wait()

                b_lo = (pad // 8) * 8
                b_hi = ((pad + cnt + 7) // 8) * 8

                def zero_row(t, carry):
                    for j in range(nhid // 16):
                        rows_v[t, pl.ds(16 * j, 16)] = zero16
                    return carry

                lax.fori_loop(b_lo, pad, zero_row, 0)
                lax.fori_loop(pad + cnt, b_hi, zero_row, 0)

                def blk(i, carry):
                    a0c, a1c, a2c, a3c = carry
                    base = b_lo + i * 8
                    for r in range(8):
                        row = base + r
                        a0c = a0c + rows_v[row, pl.ds(0, 16)]
                        a1c = a1c + rows_v[row, pl.ds(16, 16)]
                        a2c = a2c + rows_v[row, pl.ds(32, 16)]
                        a3c = a3c + rows_v[row, pl.ds(48, 16)]
                    return a0c, a1c, a2c, a3c

                accs = lax.fori_loop(0, (b_hi - b_lo) // 8, blk,
                                     (zero16, zero16, zero16, zero16))
                for j in range(nhid // 16):
                    sl = pl.ds(16 * j, 16)
                    acc_v[sl] = acc_v[sl] + accs[j]

        pltpu.sync_copy(acc_v, out_hbm.at[cid, sid])

    return k(lengths, indices, table_p)


def _tc_finalize(partials, lengths, W, b):
    B = lengths.shape[0]
    NC = W.shape[0]

    def body(p_ref, l_ref, w_ref, b_ref, o_ref):
        sums = p_ref[0] + p_ref[1]
        lv = l_ref[0].astype(jnp.float32)
        inv = jnp.where(lv > 0, 1.0 / jnp.maximum(lv, 1.0), 0.0)
        means = sums * inv[:, None]
        t = jnp.tanh(means)
        o_ref[...] = lax.dot_general(
            t, w_ref[...], (((1,), (1,)), ((), ())),
            preferred_element_type=jnp.float32) + b_ref[...]

    return pl.pallas_call(
        body,
        out_shape=jax.ShapeDtypeStruct((B, NC), jnp.float32),
    )(partials, lengths.reshape(1, B), W, b.reshape(1, NC))


def kernel(lengths, indices, table, W, b):
    nhid = table.shape[1]
    table_p = jnp.pad(table, ((0, 0), (0, 128 - nhid)))
    partials = _sc_partial_sums(lengths, indices, table_p, nhid)
    return _tc_finalize(partials, lengths, W, b)


# speedup vs baseline: 1.7187x; 1.5266x over previous
import functools

import jax
import jax.numpy as jnp
from jax import lax
from jax.experimental import pallas as pl
from jax.experimental.pallas import tpu as pltpu
from jax.experimental.pallas import tpu_sc as plsc

CH = 512
PAD = 16
CHP = CH + PAD


def _sc_partial_sums(lengths, indices, table):
    B = lengths.shape[0]
    TOT = indices.shape[0]
    NHID = table.shape[1]
    MAXLEN = TOT // B
    max_half = (MAXLEN + 1) // 2
    n_chunks = (max_half + CH - 1) // CH

    mesh = plsc.VectorSubcoreMesh(core_axis_name="c", subcore_axis_name="s")

    @functools.partial(
        pl.kernel,
        mesh=mesh,
        out_type=jax.ShapeDtypeStruct((2, B, NHID), jnp.float32),
        scratch_types=[
            pltpu.VMEM((B,), jnp.int32),
            pltpu.VMEM((CHP,), jnp.int32),
            pltpu.VMEM((CHP, NHID), jnp.float32),
            pltpu.VMEM((NHID,), jnp.float32),
            pltpu.SemaphoreType.DMA,
        ],
    )
    def k(len_hbm, idx_hbm, tab_hbm, out_hbm, len_v, idx_v, rows_v, acc_v, sem):
        cid = lax.axis_index("c")
        sid = lax.axis_index("s")

        pltpu.sync_copy(len_hbm, len_v)
        lv = len_v[...]
        seg_start = jnp.int32(0)
        seg_len = jnp.int32(0)
        run = jnp.int32(0)
        for j in range(B):
            lj = lv[j]
            seg_start = jnp.where(sid == j, run, seg_start)
            seg_len = jnp.where(sid == j, lj, seg_len)
            run = run + lj
        half0 = seg_len // 2
        my_start = seg_start + jnp.where(cid == 0, 0, half0)
        my_count = jnp.where(cid == 0, half0, seg_len - half0)

        zero16 = jnp.zeros((16,), jnp.float32)
        for j in range(NHID // 16):
            acc_v[pl.ds(16 * j, 16)] = zero16

        for k_idx in range(n_chunks):
            cnt = jnp.clip(my_count - k_idx * CH, 0, CH)

            @pl.when(cnt > 0)
            def _():
                g0 = my_start + k_idx * CH
                a0 = jnp.minimum(g0, TOT - CHP)
                a0 = pl.multiple_of((a0 // 8) * 8, 8)
                pad = g0 - a0
                pltpu.sync_copy(idx_hbm.at[pl.ds(a0, CHP)], idx_v)

                def fire(g, carry):
                    v = idx_v[pl.ds(16 * g, 16)]
                    base = 16 * g
                    for l in range(16):
                        pltpu.async_copy(
                            tab_hbm.at[v[l]], rows_v.at[base + l], sem)
                    return carry

                lax.fori_loop(0, CHP // 16, fire, 0)
                pltpu.make_async_copy(
                    tab_hbm.at[pl.ds(0, CHP)], rows_v, sem).wait()

                b_lo = (pad // 8) * 8
                b_hi = ((pad + cnt + 7) // 8) * 8

                def zero_row(t, carry):
                    for j in range(NHID // 16):
                        rows_v[t, pl.ds(16 * j, 16)] = zero16
                    return carry

                lax.fori_loop(b_lo, pad, zero_row, 0)
                lax.fori_loop(pad + cnt, b_hi, zero_row, 0)

                def blk(i, carry):
                    a0c, a1c, a2c, a3c = carry
                    base = b_lo + i * 8
                    for r in range(8):
                        row = base + r
                        a0c = a0c + rows_v[row, pl.ds(0, 16)]
                        a1c = a1c + rows_v[row, pl.ds(16, 16)]
                        a2c = a2c + rows_v[row, pl.ds(32, 16)]
                        a3c = a3c + rows_v[row, pl.ds(48, 16)]
                    return a0c, a1c, a2c, a3c

                accs = lax.fori_loop(0, (b_hi - b_lo) // 8, blk,
                                     (zero16, zero16, zero16, zero16))
                for j in range(NHID // 16):
                    sl = pl.ds(16 * j, 16)
                    acc_v[sl] = acc_v[sl] + accs[j]

        pltpu.sync_copy(acc_v, out_hbm.at[cid, sid])

    return k(lengths, indices, table)


def _tc_finalize(partials, lengths, W, b):
    B = lengths.shape[0]
    NC = W.shape[0]

    def body(p_ref, l_ref, w_ref, b_ref, o_ref):
        sums = p_ref[0] + p_ref[1]
        lv = l_ref[0].astype(jnp.float32)
        inv = jnp.where(lv > 0, 1.0 / jnp.maximum(lv, 1.0), 0.0)
        means = sums * inv[:, None]
        t = jnp.tanh(means)
        o_ref[...] = lax.dot_general(
            t, w_ref[...], (((1,), (1,)), ((), ())),
            preferred_element_type=jnp.float32) + b_ref[...]

    return pl.pallas_call(
        body,
        out_shape=jax.ShapeDtypeStruct((B, NC), jnp.float32),
    )(partials, lengths.reshape(1, B), W, b.reshape(1, NC))


def kernel(lengths, indices, table, W, b):
    partials = _sc_partial_sums(lengths, indices, table)
    return _tc_finalize(partials, lengths, W, b)


# speedup vs baseline: 2.4969x; 1.4528x over previous
import functools

import jax
import jax.numpy as jnp
from jax import lax
from jax.experimental import pallas as pl
from jax.experimental.pallas import tpu as pltpu
from jax.experimental.pallas import tpu_sc as plsc

CH = 512
PAD = 16
CHP = CH + PAD


def _sc_partial_sums(lengths, indices, table3):
    B = lengths.shape[0]
    TOT = indices.shape[0]
    NHID = table3.shape[2]
    HALF = table3.shape[1]
    MAXLEN = TOT // B
    max_half = (MAXLEN + 1) // 2
    n_chunks = (max_half + CH - 1) // CH

    mesh = plsc.VectorSubcoreMesh(core_axis_name="c", subcore_axis_name="s")

    @functools.partial(
        pl.kernel,
        mesh=mesh,
        out_type=jax.ShapeDtypeStruct((2, B, NHID), jnp.float32),
        scratch_types=[
            pltpu.VMEM((B,), jnp.int32),
            pltpu.VMEM((CHP,), jnp.int32),
            pltpu.VMEM((CHP, NHID), jnp.float32),
            pltpu.VMEM((NHID,), jnp.float32),
            pltpu.SemaphoreType.DMA,
        ],
    )
    def k(len_hbm, idx_hbm, tab_hbm, out_hbm, len_v, idx_v, rows_v, acc_v, sem):
        cid = lax.axis_index("c")
        sid = lax.axis_index("s")

        pltpu.sync_copy(len_hbm, len_v)
        lv = len_v[...]
        seg_start = jnp.int32(0)
        seg_len = jnp.int32(0)
        run = jnp.int32(0)
        for j in range(B):
            lj = lv[j]
            seg_start = jnp.where(sid == j, run, seg_start)
            seg_len = jnp.where(sid == j, lj, seg_len)
            run = run + lj
        half0 = seg_len // 2
        my_start = seg_start + jnp.where(cid == 0, 0, half0)
        my_count = jnp.where(cid == 0, half0, seg_len - half0)

        zero16 = jnp.zeros((16,), jnp.float32)
        for j in range(NHID // 16):
            acc_v[pl.ds(16 * j, 16)] = zero16

        for k_idx in range(n_chunks):
            cnt = jnp.clip(my_count - k_idx * CH, 0, CH)

            @pl.when(cnt > 0)
            def _():
                g0 = my_start + k_idx * CH
                a0 = jnp.minimum(g0, TOT - CHP)
                a0 = pl.multiple_of((a0 // 8) * 8, 8)
                pad = g0 - a0
                pltpu.sync_copy(idx_hbm.at[pl.ds(a0, CHP)], idx_v)

                def fire(g, carry):
                    v = idx_v[pl.ds(16 * g, 16)]
                    base = 16 * g
                    for l in range(16):
                        r = v[l]
                        h = jnp.where(r >= HALF, 1, 0)
                        rr = r - h * HALF
                        pltpu.async_copy(
                            tab_hbm.at[h, rr], rows_v.at[base + l], sem)
                    return carry

                lax.fori_loop(0, CHP // 16, fire, 0)
                pltpu.make_async_copy(
                    tab_hbm.at[0].at[pl.ds(0, CHP)], rows_v, sem).wait()

                b_lo = (pad // 8) * 8
                b_hi = ((pad + cnt + 7) // 8) * 8

                def zero_row(t, carry):
                    for j in range(NHID // 16):
                        rows_v[t, pl.ds(16 * j, 16)] = zero16
                    return carry

                lax.fori_loop(b_lo, pad, zero_row, 0)
                lax.fori_loop(pad + cnt, b_hi, zero_row, 0)

                def blk(i, carry):
                    a0c, a1c, a2c, a3c = carry
                    base = b_lo + i * 8
                    for r in range(8):
                        row = base + r
                        a0c = a0c + rows_v[row, pl.ds(0, 16)]
                        a1c = a1c + rows_v[row, pl.ds(16, 16)]
                        a2c = a2c + rows_v[row, pl.ds(32, 16)]
                        a3c = a3c + rows_v[row, pl.ds(48, 16)]
                    return a0c, a1c, a2c, a3c

                accs = lax.fori_loop(0, (b_hi - b_lo) // 8, blk,
                                     (zero16, zero16, zero16, zero16))
                for j in range(NHID // 16):
                    sl = pl.ds(16 * j, 16)
                    acc_v[sl] = acc_v[sl] + accs[j]

        pltpu.sync_copy(acc_v, out_hbm.at[cid, sid])

    return k(lengths, indices, table3)


def _tc_finalize(partials, lengths, W, b):
    B = lengths.shape[0]
    NC = W.shape[0]

    def body(p_ref, l_ref, w_ref, b_ref, o_ref):
        sums = p_ref[0] + p_ref[1]
        lv = l_ref[0].astype(jnp.float32)
        inv = jnp.where(lv > 0, 1.0 / jnp.maximum(lv, 1.0), 0.0)
        means = sums * inv[:, None]
        t = jnp.tanh(means)
        o_ref[...] = lax.dot_general(
            t, w_ref[...], (((1,), (1,)), ((), ())),
            preferred_element_type=jnp.float32) + b_ref[...]

    return pl.pallas_call(
        body,
        out_shape=jax.ShapeDtypeStruct((B, NC), jnp.float32),
    )(partials, lengths.reshape(1, B), W, b.reshape(1, NC))


def kernel(lengths, indices, table, W, b):
    nt = table.shape[0]
    table3 = table.reshape(2, nt // 2, table.shape[1])
    partials = _sc_partial_sums(lengths, indices, table3)
    return _tc_finalize(partials, lengths, W, b)


# speedup vs baseline: 2.5622x; 1.0262x over previous
import functools

import jax
import jax.numpy as jnp
from jax import lax
from jax.experimental import pallas as pl
from jax.experimental.pallas import tpu as pltpu
from jax.experimental.pallas import tpu_sc as plsc

CH = 512
PAD = 16
CHP = CH + PAD
NW = 32


def _sc_partial_sums(lengths, indices, table3):
    B = lengths.shape[0]
    TOT = indices.shape[0]
    NHID = table3.shape[2]
    HALF = table3.shape[1]
    MAXLEN = TOT // B
    max_per_tile = (B * (MAXLEN - 1) + NW - 1) // NW
    n_chunks = (max_per_tile + CH - 1) // CH

    mesh = plsc.VectorSubcoreMesh(core_axis_name="c", subcore_axis_name="s")

    @functools.partial(
        pl.kernel,
        mesh=mesh,
        out_type=jax.ShapeDtypeStruct((NW, B, NHID), jnp.float32),
        scratch_types=[
            pltpu.VMEM((B,), jnp.int32),
            pltpu.VMEM((CHP,), jnp.int32),
            pltpu.VMEM((CHP, NHID), jnp.float32),
            pltpu.VMEM((B, NHID), jnp.float32),
            pltpu.SemaphoreType.DMA,
        ],
    )
    def k(len_hbm, idx_hbm, tab_hbm, out_hbm, len_v, idx_v, rows_v, acc_v, sem):
        cid = lax.axis_index("c")
        sid = lax.axis_index("s")
        wid = cid * 16 + sid

        pltpu.sync_copy(len_hbm, len_v)
        lv = len_v[...]
        ends = []
        run = jnp.int32(0)
        for j in range(B):
            run = run + lv[j]
            ends.append(run)
        total = run
        q = (total + NW - 1) // NW
        my_start = jnp.minimum(wid * q, total)
        my_count = jnp.minimum(my_start + q, total) - my_start

        zero16 = jnp.zeros((16,), jnp.float32)
        for j in range(B):
            for u in range(NHID // 16):
                acc_v[j, pl.ds(16 * u, 16)] = zero16

        for k_idx in range(n_chunks):
            cnt = jnp.clip(my_count - k_idx * CH, 0, CH)

            @pl.when(cnt > 0)
            def _():
                g0 = my_start + k_idx * CH
                a0 = jnp.minimum(g0, TOT - CHP)
                a0 = pl.multiple_of((a0 // 8) * 8, 8)
                pad = g0 - a0
                pltpu.sync_copy(idx_hbm.at[pl.ds(a0, CHP)], idx_v)

                def fire(g, carry):
                    v = idx_v[pl.ds(16 * g, 16)]
                    base = 16 * g
                    for l in range(16):
                        r = v[l]
                        h = jnp.where(r >= HALF, 1, 0)
                        rr = r - h * HALF
                        pltpu.async_copy(
                            tab_hbm.at[h, rr], rows_v.at[base + l], sem)
                    return carry

                lax.fori_loop(0, CHP // 16, fire, 0)
                pltpu.make_async_copy(
                    tab_hbm.at[0].at[pl.ds(0, CHP)], rows_v, sem).wait()

                t_end = pad + cnt

                def run_body(_, t):
                    p = g0 + (t - pad)
                    seg = jnp.int32(0)
                    for e in ends:
                        seg = seg + jnp.where(p >= e, 1, 0)
                    seg = jnp.minimum(seg, B - 1)
                    seg_end = jnp.int32(0)
                    for j, e in enumerate(ends):
                        seg_end = jnp.where(seg == j, e, seg_end)
                    nxt = jnp.minimum(t_end, pad + (seg_end - g0))
                    nxt = jnp.maximum(nxt, t)
                    mid_lo = jnp.minimum(((t + 7) // 8) * 8, nxt)
                    mid_n = (nxt - mid_lo) // 8
                    mid_hi = mid_lo + mid_n * 8

                    def row_add(t2, carry):
                        a0c, a1c, a2c, a3c = carry
                        a0c = a0c + rows_v[t2, pl.ds(0, 16)]
                        a1c = a1c + rows_v[t2, pl.ds(16, 16)]
                        a2c = a2c + rows_v[t2, pl.ds(32, 16)]
                        a3c = a3c + rows_v[t2, pl.ds(48, 16)]
                        return a0c, a1c, a2c, a3c

                    def blk(i, carry):
                        a0c, a1c, a2c, a3c = carry
                        base = mid_lo + i * 8
                        for r in range(8):
                            row = base + r
                            a0c = a0c + rows_v[row, pl.ds(0, 16)]
                            a1c = a1c + rows_v[row, pl.ds(16, 16)]
                            a2c = a2c + rows_v[row, pl.ds(32, 16)]
                            a3c = a3c + rows_v[row, pl.ds(48, 16)]
                        return a0c, a1c, a2c, a3c

                    z4 = (zero16, zero16, zero16, zero16)
                    accs = lax.fori_loop(t, mid_lo, row_add, z4)
                    accs = lax.fori_loop(0, mid_n, blk, accs)
                    accs = lax.fori_loop(mid_hi, nxt, row_add, accs)
                    for u in range(NHID // 16):
                        sl = pl.ds(16 * u, 16)
                        acc_v[seg, sl] = acc_v[seg, sl] + accs[u]
                    return nxt

                lax.fori_loop(0, B, run_body, pad)

        pltpu.sync_copy(acc_v, out_hbm.at[wid])

    return k(lengths, indices, table3)


def _tc_finalize(partials, lengths, W, b):
    B = lengths.shape[0]
    NC = W.shape[0]

    def body(p_ref, l_ref, w_ref, b_ref, o_ref):
        sums = jnp.sum(p_ref[...], axis=0)
        lv = l_ref[0].astype(jnp.float32)
        inv = jnp.where(lv > 0, 1.0 / jnp.maximum(lv, 1.0), 0.0)
        means = sums * inv[:, None]
        t = jnp.tanh(means)
        o_ref[...] = lax.dot_general(
            t, w_ref[...], (((1,), (1,)), ((), ())),
            preferred_element_type=jnp.float32) + b_ref[...]

    return pl.pallas_call(
        body,
        out_shape=jax.ShapeDtypeStruct((B, NC), jnp.float32),
    )(partials, lengths.reshape(1, B), W, b.reshape(1, NC))


def kernel(lengths, indices, table, W, b):
    nt = table.shape[0]
    table3 = table.reshape(2, nt // 2, table.shape[1])
    partials = _sc_partial_sums(lengths, indices, table3)
    return _tc_finalize(partials, lengths, W, b)
